# baseline (device time: 85856 ns/iter reference)
import jax
import jax.numpy as jnp
from jax import lax
from jax.experimental import pallas as pl
from jax.experimental.pallas import tpu as pltpu

N_DEV = 8
M_PER = 512
K = 4096

S_OFF = (0, 192, 352)
S_ROWS = (192, 160, 160)
MASKS = ((1, 3, 4), (3, 4, 1), (4, 1, 3))


def _off_of_slot(s, t):
    m0, m1, m2 = MASKS[s]
    return ((m0 if t & 1 else 0) ^ (m1 if t & 2 else 0)
            ^ (m2 if t & 4 else 0))


def kernel(x, w_mat, scale_x, scale_w):
    m_per, k = x.shape
    _, n_per = w_mat.shape
    assert (m_per, k) == (M_PER, K)

    dot_dims = (((1,), (0,)), ((), ()))

    def body(x_hbm, w_hbm, sx_ref, sw_ref, out_hbm, ca_ref, cb_ref,
             cc_ref, xf_ref, wf_ref, w8_ref, ob_ref, local_sems,
             out_sems, send_sems, recv_sems, recv3_sems):
        my = lax.axis_index("i")
        comm = (ca_ref, cb_ref, cc_ref)

        def slot_ref(s, slot, nslots=1):
            return comm[s].at[pl.ds(slot * S_ROWS[s], nslots * S_ROWS[s]), :]

        cp_x = []
        for s in range(3):
            cp = pltpu.make_async_copy(
                x_hbm.at[pl.ds(S_OFF[s], S_ROWS[s]), :],
                xf_ref.at[pl.ds(S_OFF[s], S_ROWS[s]), :],
                local_sems.at[s],
            )
            cp.start()
            cp_x.append(cp)

        barrier_sem = pltpu.get_barrier_semaphore()
        for mask in (1, 3, 4):
            pl.semaphore_signal(
                barrier_sem, inc=1,
                device_id=(my ^ mask,), device_id_type=pl.DeviceIdType.MESH,
            )
        pl.semaphore_wait(barrier_sem, 3)

        scale = sx_ref[0] * sw_ref[0]

        def make_rdma(s, r, slot, j=None):
            recv_sem = (
                recv3_sems.at[s, j] if r == 2 else recv_sems.at[s, r]
            )
            return pltpu.make_async_remote_copy(
                src_ref=slot_ref(s, slot),
                dst_ref=slot_ref(s, slot ^ (1 << r)),
                send_sem=send_sems.at[s, r],
                recv_sem=recv_sem,
                device_id=(my ^ MASKS[s][r],),
                device_id_type=pl.DeviceIdType.MESH,
            )

        ob_last = [0, 0]
        ob_slot = [0]

        def ob_wait(slot):
            if ob_last[slot]:
                pltpu.make_async_copy(
                    ob_ref.at[slot, pl.ds(0, ob_last[slot]), :],
                    out_hbm.at[pl.ds(0, ob_last[slot]), :],
                    out_sems.at[slot],
                ).wait()
                ob_last[slot] = 0

        def emit_rows(vals, row0, rows):
            slot = ob_slot[0]
            ob_slot[0] ^= 1
            ob_wait(slot)
            ob_ref[slot, pl.ds(0, rows), :] = vals
            pltpu.make_async_copy(
                ob_ref.at[slot, pl.ds(0, rows), :],
                out_hbm.at[pl.ds(row0, rows), :],
                out_sems.at[slot],
            ).start()
            ob_last[slot] = rows

        def group_gemm(s, slot0, nslots):
            rows = S_ROWS[s]
            acc = lax.dot_general(
                slot_ref(s, slot0, nslots)[...], w8_ref[...], dot_dims,
                preferred_element_type=jnp.float32,
            )
            vals = jnp.maximum(acc * scale, 0.0)
            for i in range(nslots):
                off = _off_of_slot(s, slot0 + i)
                emit_rows(
                    vals[i * rows:(i + 1) * rows, :],
                    (my ^ off) * M_PER + S_OFF[s],
                    rows,
                )

        r0 = []
        for s in range(3):
            cp_x[s].wait()
            comm[s][pl.ds(0, S_ROWS[s]), :] = xf_ref[
                pl.ds(S_OFF[s], S_ROWS[s]), :
            ].astype(jnp.float8_e4m3fn)
            rdma = make_rdma(s, 0, 0)
            rdma.start()
            r0.append(rdma)

        for half in range(2):
            cp_w = pltpu.make_async_copy(
                w_hbm.at[pl.ds(half * (K // 2), K // 2), :],
                wf_ref, local_sems.at[3 + half],
            )
            cp_w.start()
            cp_w.wait()
            w8_ref[pl.ds(half * (K // 2), K // 2), :] = wf_ref[...].astype(
                jnp.float8_e5m2
            )

        r1 = [[] for _ in range(3)]
        for s in (1, 2, 0):
            r0[s].wait()
            for slot in (0, 1):
                rdma = make_rdma(s, 1, slot)
                rdma.start()
                r1[s].append(rdma)

        for s in range(3):
            group_gemm(s, 0, 2)

        r2 = [[None] * 4 for _ in range(3)]
        for s in (1, 2, 0):
            for rdma in r1[s]:
                rdma.wait()
            for j in range(4):
                rdma = make_rdma(s, 2, j, j=j)
                rdma.start()
                r2[s][j] = rdma

        for s in range(3):
            group_gemm(s, 2, 2)
        for g in range(2):
            for s in (1, 2, 0):
                r2[s][2 * g].wait()
                r2[s][2 * g + 1].wait()
                group_gemm(s, 4 + 2 * g, 2)

        for slot in range(2):
            ob_wait(slot)

    return pl.pallas_call(
        body,
        out_shape=jax.ShapeDtypeStruct((N_DEV * m_per, n_per), jnp.float32),
        in_specs=[
            pl.BlockSpec(memory_space=pl.ANY),
            pl.BlockSpec(memory_space=pl.ANY),
            pl.BlockSpec(memory_space=pltpu.SMEM),
            pl.BlockSpec(memory_space=pltpu.SMEM),
        ],
        out_specs=pl.BlockSpec(memory_space=pl.ANY),
        scratch_shapes=[
            pltpu.VMEM((N_DEV * S_ROWS[0], K), jnp.float8_e4m3fn),
            pltpu.VMEM((N_DEV * S_ROWS[1], K), jnp.float8_e4m3fn),
            pltpu.VMEM((N_DEV * S_ROWS[2], K), jnp.float8_e4m3fn),
            pltpu.VMEM((M_PER, K), jnp.float32),
            pltpu.VMEM((K // 2, n_per), jnp.float32),
            pltpu.VMEM((K, n_per), jnp.float8_e5m2),
            pltpu.VMEM((2, M_PER, n_per), jnp.float32),
            pltpu.SemaphoreType.DMA((5,)),
            pltpu.SemaphoreType.DMA((2,)),
            pltpu.SemaphoreType.DMA((3, 3)),
            pltpu.SemaphoreType.DMA((3, 2)),
            pltpu.SemaphoreType.DMA((3, 4)),
        ],
        compiler_params=pltpu.CompilerParams(
            collective_id=0,
            vmem_limit_bytes=100 * 1024 * 1024,
        ),
    )(x, w_mat, scale_x, scale_w)


# device time: 82781 ns/iter; 1.0371x vs baseline; 1.0371x over previous
import jax
import jax.numpy as jnp
from jax import lax
from jax.experimental import pallas as pl
from jax.experimental.pallas import tpu as pltpu

N_DEV = 8
M_PER = 512
K = 4096

K_OFF = (0, 1408, 2816)
K_SZ = (1408, 1408, 1280)
MASKS = ((1, 3, 4), (3, 4, 1), (4, 1, 3))


def _off_of_slot(s, t):
    m0, m1, m2 = MASKS[s]
    return ((m0 if t & 1 else 0) ^ (m1 if t & 2 else 0)
            ^ (m2 if t & 4 else 0))


def kernel(x, w_mat, scale_x, scale_w):
    m_per, k = x.shape
    _, n_per = w_mat.shape
    assert (m_per, k) == (M_PER, K)

    dot_dims = (((1,), (0,)), ((), ()))

    def body(x_hbm, w_hbm, sx_ref, sw_ref, out_hbm, ca_ref, cb_ref,
             cc_ref, xf_ref, wf_ref, w8_ref, acc_ref, ob_ref,
             local_sems, out_sems, send_sems, recv_sems, recv3_sems):
        my = lax.axis_index("i")
        comm = (ca_ref, cb_ref, cc_ref)

        def slot_ref(s, slot):
            return comm[s].at[pl.ds(slot * M_PER, M_PER), :]

        cp_x = []
        for s in range(3):
            cp = pltpu.make_async_copy(
                x_hbm.at[:, pl.ds(K_OFF[s], K_SZ[s])],
                xf_ref.at[:, pl.ds(K_OFF[s], K_SZ[s])],
                local_sems.at[s],
            )
            cp.start()
            cp_x.append(cp)

        barrier_sem = pltpu.get_barrier_semaphore()
        for mask in (1, 3, 4):
            pl.semaphore_signal(
                barrier_sem, inc=1,
                device_id=(my ^ mask,), device_id_type=pl.DeviceIdType.MESH,
            )
        pl.semaphore_wait(barrier_sem, 3)

        scale = sx_ref[0] * sw_ref[0]

        def make_rdma(s, r, slot, j=None):
            recv_sem = (
                recv3_sems.at[s, j] if r == 2 else recv_sems.at[s, r]
            )
            return pltpu.make_async_remote_copy(
                src_ref=slot_ref(s, slot),
                dst_ref=slot_ref(s, slot ^ (1 << r)),
                send_sem=send_sems.at[s, r],
                recv_sem=recv_sem,
                device_id=(my ^ MASKS[s][r],),
                device_id_type=pl.DeviceIdType.MESH,
            )

        ob_busy = [False, False]
        ob_slot = [0]

        def ob_wait(slot):
            if ob_busy[slot]:
                pltpu.make_async_copy(
                    ob_ref.at[slot],
                    out_hbm.at[pl.ds(0, M_PER), :],
                    out_sems.at[slot],
                ).wait()
                ob_busy[slot] = False

        def emit_chunk(vals, row0):
            slot = ob_slot[0]
            ob_slot[0] ^= 1
            ob_wait(slot)
            ob_ref[slot] = vals
            pltpu.make_async_copy(
                ob_ref.at[slot],
                out_hbm.at[pl.ds(row0, M_PER), :],
                out_sems.at[slot],
            ).start()
            ob_busy[slot] = True

        counts = {}

        def partial(s, slot):
            off = _off_of_slot(s, slot)
            part = lax.dot_general(
                comm[s][pl.ds(slot * M_PER, M_PER), :],
                w8_ref[pl.ds(K_OFF[s], K_SZ[s]), :],
                dot_dims, preferred_element_type=jnp.float32,
            )
            arows = pl.ds(off * M_PER, M_PER)
            c = counts.get(off, 0)
            counts[off] = c + 1
            if c == 0:
                acc_ref[arows, :] = part
            else:
                acc_ref[arows, :] = acc_ref[arows, :] + part
            if c + 1 == 3:
                vals = jnp.maximum(acc_ref[arows, :] * scale, 0.0)
                emit_chunk(vals, (my ^ off) * M_PER)

        r0 = []
        for s in range(3):
            cp_x[s].wait()
            comm[s][pl.ds(0, M_PER), :] = xf_ref[
                :, pl.ds(K_OFF[s], K_SZ[s])
            ].astype(jnp.float8_e4m3fn)
            rdma = make_rdma(s, 0, 0)
            rdma.start()
            r0.append(rdma)

        for half in range(2):
            cp_w = pltpu.make_async_copy(
                w_hbm.at[pl.ds(half * (K // 2), K // 2), :],
                wf_ref, local_sems.at[3 + half],
            )
            cp_w.start()
            cp_w.wait()
            w8_ref[pl.ds(half * (K // 2), K // 2), :] = wf_ref[...].astype(
                jnp.float8_e5m2
            )
            if half == 0:
                partial(0, 0)
        partial(1, 0)
        partial(2, 0)

        r1 = [[] for _ in range(3)]
        for s in (2, 0, 1):
            r0[s].wait()
            for slot in (0, 1):
                rdma = make_rdma(s, 1, slot)
                rdma.start()
                r1[s].append(rdma)

        for s in range(3):
            partial(s, 1)

        r2 = [[None] * 4 for _ in range(3)]
        for s in (2, 0, 1):
            for rdma in r1[s]:
                rdma.wait()
            for j in range(4):
                rdma = make_rdma(s, 2, j, j=j)
                rdma.start()
                r2[s][j] = rdma

        for s in range(3):
            partial(s, 2)
            partial(s, 3)
        for j in range(4):
            for s in (2, 0, 1):
                r2[s][j].wait()
                partial(s, 4 + j)

        for slot in range(2):
            ob_wait(slot)

    return pl.pallas_call(
        body,
        out_shape=jax.ShapeDtypeStruct((N_DEV * m_per, n_per), jnp.float32),
        in_specs=[
            pl.BlockSpec(memory_space=pl.ANY),
            pl.BlockSpec(memory_space=pl.ANY),
            pl.BlockSpec(memory_space=pltpu.SMEM),
            pl.BlockSpec(memory_space=pltpu.SMEM),
        ],
        out_specs=pl.BlockSpec(memory_space=pl.ANY),
        scratch_shapes=[
            pltpu.VMEM((N_DEV * M_PER, K_SZ[0]), jnp.float8_e4m3fn),
            pltpu.VMEM((N_DEV * M_PER, K_SZ[1]), jnp.float8_e4m3fn),
            pltpu.VMEM((N_DEV * M_PER, K_SZ[2]), jnp.float8_e4m3fn),
            pltpu.VMEM((M_PER, K), jnp.float32),
            pltpu.VMEM((K // 2, n_per), jnp.float32),
            pltpu.VMEM((K, n_per), jnp.float8_e5m2),
            pltpu.VMEM((N_DEV * M_PER, n_per), jnp.float32),
            pltpu.VMEM((2, M_PER, n_per), jnp.float32),
            pltpu.SemaphoreType.DMA((5,)),
            pltpu.SemaphoreType.DMA((2,)),
            pltpu.SemaphoreType.DMA((3, 3)),
            pltpu.SemaphoreType.DMA((3, 2)),
            pltpu.SemaphoreType.DMA((3, 4)),
        ],
        compiler_params=pltpu.CompilerParams(
            collective_id=0,
            vmem_limit_bytes=100 * 1024 * 1024,
        ),
    )(x, w_mat, scale_x, scale_w)


# device time: 82727 ns/iter; 1.0378x vs baseline; 1.0007x over previous
import jax
import jax.numpy as jnp
from jax import lax
from jax.experimental import pallas as pl
from jax.experimental.pallas import tpu as pltpu

N_DEV = 8
M_PER = 512
K = 4096

K_OFF = (0, 1408, 2816)
K_SZ = (1408, 1408, 1280)
MASKS = ((1, 3, 4), (3, 4, 1), (4, 1, 3))


def _off_of_slot(s, t):
    m0, m1, m2 = MASKS[s]
    return ((m0 if t & 1 else 0) ^ (m1 if t & 2 else 0)
            ^ (m2 if t & 4 else 0))


def kernel(x, w_mat, scale_x, scale_w):
    m_per, k = x.shape
    _, n_per = w_mat.shape
    assert (m_per, k) == (M_PER, K)

    dot_dims = (((1,), (0,)), ((), ()))

    def body(x_hbm, w_hbm, sx_ref, sw_ref, out_hbm, ca_ref, cb_ref,
             cc_ref, xf_ref, wf_ref, w8_ref, acc_ref, ob_ref,
             local_sems, out_sems, send_sems, recv_sems, recv3_sems):
        my = lax.axis_index("i")
        comm = (ca_ref, cb_ref, cc_ref)

        def slot_ref(s, slot):
            return comm[s].at[pl.ds(slot * M_PER, M_PER), :]

        cp_x = []
        for s in range(3):
            cp = pltpu.make_async_copy(
                x_hbm.at[:, pl.ds(K_OFF[s], K_SZ[s])],
                xf_ref.at[:, pl.ds(K_OFF[s], K_SZ[s])],
                local_sems.at[s],
            )
            cp.start()
            cp_x.append(cp)

        barrier_sem = pltpu.get_barrier_semaphore()
        for mask in (1, 3, 4):
            pl.semaphore_signal(
                barrier_sem, inc=1,
                device_id=(my ^ mask,), device_id_type=pl.DeviceIdType.MESH,
            )
        pl.semaphore_wait(barrier_sem, 3)

        scale = sx_ref[0] * sw_ref[0]

        def make_rdma(s, r, slot, j=None):
            recv_sem = (
                recv3_sems.at[s, j] if r == 2 else recv_sems.at[s, r]
            )
            return pltpu.make_async_remote_copy(
                src_ref=slot_ref(s, slot),
                dst_ref=slot_ref(s, slot ^ (1 << r)),
                send_sem=send_sems.at[s, r],
                recv_sem=recv_sem,
                device_id=(my ^ MASKS[s][r],),
                device_id_type=pl.DeviceIdType.MESH,
            )

        ob_busy = [False, False]
        ob_slot = [0]

        def ob_wait(slot):
            if ob_busy[slot]:
                pltpu.make_async_copy(
                    ob_ref.at[slot],
                    out_hbm.at[pl.ds(0, M_PER), :],
                    out_sems.at[slot],
                ).wait()
                ob_busy[slot] = False

        def emit_chunk(vals, row0):
            slot = ob_slot[0]
            ob_slot[0] ^= 1
            ob_wait(slot)
            ob_ref[slot] = vals
            pltpu.make_async_copy(
                ob_ref.at[slot],
                out_hbm.at[pl.ds(row0, M_PER), :],
                out_sems.at[slot],
            ).start()
            ob_busy[slot] = True

        counts = {}

        def partial(s, slot):
            off = _off_of_slot(s, slot)
            part = lax.dot_general(
                comm[s][pl.ds(slot * M_PER, M_PER), :],
                w8_ref[pl.ds(K_OFF[s], K_SZ[s]), :],
                dot_dims, preferred_element_type=jnp.float32,
            )
            arows = pl.ds(off * M_PER, M_PER)
            c = counts.get(off, 0)
            counts[off] = c + 1
            if c == 0:
                acc_ref[arows, :] = part
            else:
                acc_ref[arows, :] = acc_ref[arows, :] + part
            if c + 1 == 3:
                vals = jnp.maximum(acc_ref[arows, :] * scale, 0.0)
                emit_chunk(vals, (my ^ off) * M_PER)

        r0 = []
        for s in range(3):
            cp_x[s].wait()
            comm[s][pl.ds(0, M_PER), :] = xf_ref[
                :, pl.ds(K_OFF[s], K_SZ[s])
            ].astype(jnp.float8_e4m3fn)
            rdma = make_rdma(s, 0, 0)
            rdma.start()
            r0.append(rdma)

        for half in range(2):
            cp_w = pltpu.make_async_copy(
                w_hbm.at[pl.ds(half * (K // 2), K // 2), :],
                wf_ref, local_sems.at[3 + half],
            )
            cp_w.start()
            cp_w.wait()
            w8_ref[pl.ds(half * (K // 2), K // 2), :] = wf_ref[...].astype(
                jnp.float8_e4m3fn
            )
            if half == 0:
                partial(0, 0)
        partial(1, 0)
        partial(2, 0)

        r1 = [[] for _ in range(3)]
        for s in (2, 0, 1):
            r0[s].wait()
            for slot in (0, 1):
                rdma = make_rdma(s, 1, slot)
                rdma.start()
                r1[s].append(rdma)

        for s in range(3):
            partial(s, 1)

        r2 = [[None] * 4 for _ in range(3)]
        for s in (2, 0, 1):
            for rdma in r1[s]:
                rdma.wait()
            for j in range(4):
                rdma = make_rdma(s, 2, j, j=j)
                rdma.start()
                r2[s][j] = rdma

        for s in range(3):
            partial(s, 2)
            partial(s, 3)
        for j in range(4):
            for s in (2, 0, 1):
                r2[s][j].wait()
                partial(s, 4 + j)

        for slot in range(2):
            ob_wait(slot)

    return pl.pallas_call(
        body,
        out_shape=jax.ShapeDtypeStruct((N_DEV * m_per, n_per), jnp.float32),
        in_specs=[
            pl.BlockSpec(memory_space=pl.ANY),
            pl.BlockSpec(memory_space=pl.ANY),
            pl.BlockSpec(memory_space=pltpu.SMEM),
            pl.BlockSpec(memory_space=pltpu.SMEM),
        ],
        out_specs=pl.BlockSpec(memory_space=pl.ANY),
        scratch_shapes=[
            pltpu.VMEM((N_DEV * M_PER, K_SZ[0]), jnp.float8_e4m3fn),
            pltpu.VMEM((N_DEV * M_PER, K_SZ[1]), jnp.float8_e4m3fn),
            pltpu.VMEM((N_DEV * M_PER, K_SZ[2]), jnp.float8_e4m3fn),
            pltpu.VMEM((M_PER, K), jnp.float32),
            pltpu.VMEM((K // 2, n_per), jnp.float32),
            pltpu.VMEM((K, n_per), jnp.float8_e4m3fn),
            pltpu.VMEM((N_DEV * M_PER, n_per), jnp.float32),
            pltpu.VMEM((2, M_PER, n_per), jnp.float32),
            pltpu.SemaphoreType.DMA((5,)),
            pltpu.SemaphoreType.DMA((2,)),
            pltpu.SemaphoreType.DMA((3, 3)),
            pltpu.SemaphoreType.DMA((3, 2)),
            pltpu.SemaphoreType.DMA((3, 4)),
        ],
        compiler_params=pltpu.CompilerParams(
            collective_id=0,
            vmem_limit_bytes=100 * 1024 * 1024,
        ),
    )(x, w_mat, scale_x, scale_w)


# device time: 82586 ns/iter; 1.0396x vs baseline; 1.0017x over previous
import jax
import jax.numpy as jnp
from jax import lax
from jax.experimental import pallas as pl
from jax.experimental.pallas import tpu as pltpu

N_DEV = 8
M_PER = 512
K = 4096

K_OFF = (0, 1408, 2816)
K_SZ = (1408, 1408, 1280)
MASKS = ((1, 3, 4), (3, 4, 1), (4, 1, 3))


def _off_of_slot(s, t):
    m0, m1, m2 = MASKS[s]
    return ((m0 if t & 1 else 0) ^ (m1 if t & 2 else 0)
            ^ (m2 if t & 4 else 0))


def kernel(x, w_mat, scale_x, scale_w):
    m_per, k = x.shape
    _, n_per = w_mat.shape
    assert (m_per, k) == (M_PER, K)

    dot_dims = (((1,), (0,)), ((), ()))

    def body(x_hbm, w_hbm, sx_ref, sw_ref, out_hbm, ca_ref, cb_ref,
             cc_ref, xf_ref, wf_ref, w8_ref, acc_ref, ob_ref,
             local_sems, out_sems, send_sems, recv_sems, recv3_sems):
        my = lax.axis_index("i")
        comm = (ca_ref, cb_ref, cc_ref)

        def slot_ref(s, slot):
            return comm[s].at[pl.ds(slot * M_PER, M_PER), :]

        cp_x = []
        for s in range(3):
            cp = pltpu.make_async_copy(
                x_hbm.at[:, pl.ds(K_OFF[s], K_SZ[s])],
                xf_ref.at[:, pl.ds(K_OFF[s], K_SZ[s])],
                local_sems.at[s],
            )
            cp.start()
            cp_x.append(cp)

        barrier_sem = pltpu.get_barrier_semaphore()
        for mask in (1, 3, 4):
            pl.semaphore_signal(
                barrier_sem, inc=1,
                device_id=(my ^ mask,), device_id_type=pl.DeviceIdType.MESH,
            )
        pl.semaphore_wait(barrier_sem, 3)

        scale = sx_ref[0] * sw_ref[0]

        def make_rdma(s, r, slot, j=None):
            recv_sem = (
                recv3_sems.at[s, j] if r == 2 else recv_sems.at[s, r]
            )
            return pltpu.make_async_remote_copy(
                src_ref=slot_ref(s, slot),
                dst_ref=slot_ref(s, slot ^ (1 << r)),
                send_sem=send_sems.at[s, r],
                recv_sem=recv_sem,
                device_id=(my ^ MASKS[s][r],),
                device_id_type=pl.DeviceIdType.MESH,
            )

        ob_busy = [False, False]
        ob_slot = [0]

        def ob_wait(slot):
            if ob_busy[slot]:
                pltpu.make_async_copy(
                    ob_ref.at[slot],
                    out_hbm.at[pl.ds(0, M_PER), :],
                    out_sems.at[slot],
                ).wait()
                ob_busy[slot] = False

        def emit_chunk(vals, row0):
            slot = ob_slot[0]
            ob_slot[0] ^= 1
            ob_wait(slot)
            ob_ref[slot] = vals
            pltpu.make_async_copy(
                ob_ref.at[slot],
                out_hbm.at[pl.ds(row0, M_PER), :],
                out_sems.at[slot],
            ).start()
            ob_busy[slot] = True

        counts = {}

        def partial(s, slot):
            off = _off_of_slot(s, slot)
            part = lax.dot_general(
                comm[s][pl.ds(slot * M_PER, M_PER), :],
                w8_ref[pl.ds(K_OFF[s], K_SZ[s]), :],
                dot_dims, preferred_element_type=jnp.float32,
            )
            arows = pl.ds(off * M_PER, M_PER)
            c = counts.get(off, 0)
            counts[off] = c + 1
            if c == 0:
                acc_ref[arows, :] = part
            else:
                acc_ref[arows, :] = acc_ref[arows, :] + part
            if c + 1 == 3:
                vals = jnp.maximum(acc_ref[arows, :] * scale, 0.0)
                emit_chunk(vals, (my ^ off) * M_PER)

        r0 = []
        for s in range(3):
            cp_x[s].wait()
            comm[s][pl.ds(0, M_PER), :] = xf_ref[
                :, pl.ds(K_OFF[s], K_SZ[s])
            ].astype(jnp.float8_e4m3fn)
            rdma = make_rdma(s, 0, 0)
            rdma.start()
            r0.append(rdma)

        for half in range(2):
            cp_w = pltpu.make_async_copy(
                w_hbm.at[pl.ds(half * (K // 2), K // 2), :],
                wf_ref, local_sems.at[3 + half],
            )
            cp_w.start()
            cp_w.wait()
            w8_ref[pl.ds(half * (K // 2), K // 2), :] = wf_ref[...].astype(
                jnp.float8_e5m2
            )
            if half == 0:
                partial(0, 0)
        partial(1, 0)
        partial(2, 0)

        r1 = [[] for _ in range(3)]
        for s in (2, 0, 1):
            r0[s].wait()
            for slot in (0, 1):
                rdma = make_rdma(s, 1, slot)
                rdma.start()
                r1[s].append(rdma)

        for s in range(3):
            partial(s, 1)

        r2 = [[None] * 4 for _ in range(3)]
        for s in (2, 0, 1):
            for rdma in r1[s]:
                rdma.wait()
            for j in range(4):
                rdma = make_rdma(s, 2, j, j=j)
                rdma.start()
                r2[s][j] = rdma

        for s in range(3):
            partial(s, 2)
            partial(s, 3)
        for j in range(4):
            for s in (2, 0, 1):
                r2[s][j].wait()
                partial(s, 4 + j)

        for slot in range(2):
            ob_wait(slot)

    return pl.pallas_call(
        body,
        out_shape=jax.ShapeDtypeStruct((N_DEV * m_per, n_per), jnp.float32),
        in_specs=[
            pl.BlockSpec(memory_space=pl.ANY),
            pl.BlockSpec(memory_space=pl.ANY),
            pl.BlockSpec(memory_space=pltpu.SMEM),
            pl.BlockSpec(memory_space=pltpu.SMEM),
        ],
        out_specs=pl.BlockSpec(memory_space=pl.ANY),
        scratch_shapes=[
            pltpu.VMEM((N_DEV * M_PER, K_SZ[0]), jnp.float8_e4m3fn),
            pltpu.VMEM((N_DEV * M_PER, K_SZ[1]), jnp.float8_e4m3fn),
            pltpu.VMEM((N_DEV * M_PER, K_SZ[2]), jnp.float8_e4m3fn),
            pltpu.VMEM((M_PER, K), jnp.float32),
            pltpu.VMEM((K // 2, n_per), jnp.float32),
            pltpu.VMEM((K, n_per), jnp.float8_e5m2),
            pltpu.VMEM((N_DEV * M_PER, n_per), jnp.float32),
            pltpu.VMEM((2, M_PER, n_per), jnp.float32),
            pltpu.SemaphoreType.DMA((5,)),
            pltpu.SemaphoreType.DMA((2,)),
            pltpu.SemaphoreType.DMA((3, 3)),
            pltpu.SemaphoreType.DMA((3, 2)),
            pltpu.SemaphoreType.DMA((3, 4)),
        ],
        compiler_params=pltpu.CompilerParams(
            collective_id=0,
            vmem_limit_bytes=100 * 1024 * 1024,
        ),
    )(x, w_mat, scale_x, scale_w)
